# speedup vs baseline: 1.0990x; 1.0990x over previous
import jax
import jax.numpy as jnp
from jax import lax
from jax.experimental import pallas as pl
from jax.experimental.pallas import tpu as pltpu
from jax.experimental.pallas import tpu_sc as plsc

N_NODES = 100000
EMBED_DIM = 256
NC = 2
NS = 16
NW = NC * NS

CHUNK = 128

ROWS_MAIN = 3128
ROWS_LAST = N_NODES - 31 * ROWS_MAIN
N_ITERS = -(-ROWS_MAIN // CHUNK)


def _gather_body(idx_hbm, table_hbm, out_hbm,
                 idx0, idx1, rows0, rows1,
                 isem0, isem1, gsem0, gsem1, osem0, osem1):
    wid = lax.axis_index("s") * NC + lax.axis_index("c")
    base = wid * ROWS_MAIN
    count = jnp.where(wid == NW - 1, ROWS_LAST, ROWS_MAIN)
    last_start = base + count - CHUNK

    idx_bufs = (idx0, idx1)
    rows_bufs = (rows0, rows1)
    isems = (isem0, isem1)
    gsems = (gsem0, gsem1)
    osems = (osem0, osem1)

    def chunk_start(j):
        return jnp.minimum(base + j * CHUNK, last_start)

    idx_loads = [None] * N_ITERS
    gathers = [None] * N_ITERS
    stores = [None] * N_ITERS
    idx_loads[0] = pltpu.make_async_copy(
        idx_hbm.at[pl.ds(chunk_start(0), CHUNK)], idx_bufs[0], isems[0])
    idx_loads[0].start()

    for j in range(N_ITERS):
        b = j % 2
        nb = (j + 1) % 2
        idx_loads[j].wait()
        if j >= 2:
            stores[j - 2].wait()
        gathers[j] = pltpu.make_async_copy(
            table_hbm.at[idx_bufs[b]], rows_bufs[b], gsems[b])
        gathers[j].start()
        if j >= 1:
            gathers[j - 1].wait()
            stores[j - 1] = pltpu.make_async_copy(
                rows_bufs[nb], out_hbm.at[pl.ds(chunk_start(j - 1), CHUNK)],
                osems[nb])
            stores[j - 1].start()
        if j + 1 < N_ITERS:
            idx_loads[j + 1] = pltpu.make_async_copy(
                idx_hbm.at[pl.ds(chunk_start(j + 1), CHUNK)], idx_bufs[nb],
                isems[nb])
            idx_loads[j + 1].start()

    j = N_ITERS - 1
    b = j % 2
    gathers[j].wait()
    stores[j] = pltpu.make_async_copy(
        rows_bufs[b], out_hbm.at[pl.ds(chunk_start(j), CHUNK)], osems[b])
    stores[j].start()
    stores[j - 1].wait()
    stores[j].wait()


@jax.jit
def _gather(node_specie, embeddings):
    mesh = plsc.VectorSubcoreMesh(
        core_axis_name="c", subcore_axis_name="s",
        num_cores=NC, num_subcores=NS)
    return pl.kernel(
        _gather_body,
        out_type=jax.ShapeDtypeStruct((N_NODES, EMBED_DIM), jnp.float32),
        mesh=mesh,
        scratch_types=[
            pltpu.VMEM((CHUNK,), jnp.int32),
            pltpu.VMEM((CHUNK,), jnp.int32),
            pltpu.VMEM((CHUNK, EMBED_DIM), jnp.float32),
            pltpu.VMEM((CHUNK, EMBED_DIM), jnp.float32),
            pltpu.SemaphoreType.DMA,
            pltpu.SemaphoreType.DMA,
            pltpu.SemaphoreType.DMA,
            pltpu.SemaphoreType.DMA,
            pltpu.SemaphoreType.DMA,
            pltpu.SemaphoreType.DMA,
        ],
        name="embedding_gather_sc",
    )(node_specie, embeddings)


def kernel(node_specie, embeddings):
    return _gather(node_specie.astype(jnp.int32), embeddings)
